# baseline (speedup 1.0000x reference)
import functools
import numpy as np
import jax
import jax.numpy as jnp
from jax.experimental import pallas as pl
from jax.experimental.pallas import tpu as pltpu
from jax.experimental.pallas import tpu_sc as plsc

B, N, S, K = 8, 4096, 1024, 32
R2 = np.float32(np.float64(0.2) ** 2)
M = B * S
NTOT = K * M
EPS = np.float32(1e-5)
SBLK = 256
HIGH = jax.lax.Precision.HIGHEST


FB = 4


def _fps_body(xyz_ref, out_ref):
    x = xyz_ref[:, 0, :]
    y = xyz_ref[:, 1, :]
    z = xyz_ref[:, 2, :]
    lane_n = jax.lax.broadcasted_iota(jnp.int32, (FB, N), 1)
    lane_s = jax.lax.broadcasted_iota(jnp.int32, (FB, S), 1)

    def step(t, carry):
        dist, far, ax, ay, az = carry
        oh = (lane_n == far).astype(jnp.float32)
        cx = jnp.sum(x * oh, axis=1, keepdims=True)
        cy = jnp.sum(y * oh, axis=1, keepdims=True)
        cz = jnp.sum(z * oh, axis=1, keepdims=True)
        ohs = (lane_s == t).astype(jnp.float32)
        ax = ax + cx * ohs
        ay = ay + cy * ohs
        az = az + cz * ohs
        dx = x - cx
        dy = y - cy
        dz = z - cz
        d = dx * dx + dy * dy + dz * dz
        dist = jnp.minimum(dist, d)
        m = jnp.max(dist, axis=1, keepdims=True)
        far = jnp.min(jnp.where(dist == m, lane_n, N), axis=1, keepdims=True)
        return dist, far, ax, ay, az

    dist0 = jnp.full((FB, N), 1e10, jnp.float32)
    far0 = jnp.zeros((FB, 1), jnp.int32)
    zS = jnp.zeros((FB, S), jnp.float32)
    _, _, ax, ay, az = jax.lax.fori_loop(0, S, step, (dist0, far0, zS, zS, zS))
    out_ref[:, 0, :] = ax
    out_ref[:, 1, :] = ay
    out_ref[:, 2, :] = az


def _fps_call(point_xyz):
    return pl.pallas_call(
        _fps_body,
        grid=(B // FB,),
        in_specs=[pl.BlockSpec((FB, 3, N), lambda i: (i, 0, 0))],
        out_specs=pl.BlockSpec((FB, 3, S), lambda i: (i, 0, 0)),
        out_shape=jax.ShapeDtypeStruct((B, 3, S), jnp.float32),
        compiler_params=pltpu.CompilerParams(
            dimension_semantics=("parallel",)),
    )(point_xyz)


def _ballq_body(xyz_ref, nxt_ref, out_ref):
    b = pl.program_id(0)
    px = xyz_ref[0]
    cn = nxt_ref[0]
    cc = jnp.sum(cn * cn, axis=1, keepdims=True)
    pp = jnp.sum(px * px, axis=0, keepdims=True)
    mm = jnp.dot(cn, px, preferred_element_type=jnp.float32)
    d2 = (cc + pp) - 2.0 * mm
    mask = (d2 <= R2).astype(jnp.bfloat16)
    nch = N // 128
    li = jax.lax.broadcasted_iota(jnp.int32, (128, 128), 0)
    lj = jax.lax.broadcasted_iota(jnp.int32, (128, 128), 1)
    ut = (li <= lj).astype(jnp.bfloat16)
    intras = [jnp.dot(mask[:, c * 128:(c + 1) * 128], ut,
                      preferred_element_type=jnp.float32)
              for c in range(nch)]
    cs = jnp.concatenate([ic[:, 127:128] for ic in intras], axis=1)
    ci = jax.lax.broadcasted_iota(jnp.int32, (nch, nch), 0)
    cj = jax.lax.broadcasted_iota(jnp.int32, (nch, nch), 1)
    uts = (ci < cj).astype(jnp.float32)
    csx = jnp.dot(cs, uts, precision=HIGH,
                  preferred_element_type=jnp.float32)
    rank = jnp.concatenate(
        [intras[c] + csx[:, c:c + 1] for c in range(nch)], axis=1)
    count = rank[:, N - 1:N]
    raws = [jnp.sum((rank <= j).astype(jnp.float32), axis=1, keepdims=True)
            for j in range(K)]
    raw = jnp.concatenate(raws, axis=1).astype(jnp.int32)
    count = count.astype(jnp.int32)
    jv = jax.lax.broadcasted_iota(jnp.int32, (SBLK, K), 1)
    idx = jnp.where(jv < count, raw, raw[:, 0:1])
    out_ref[0] = idx + b * N


def _ballq_call(point_xyz, nxt):
    return pl.pallas_call(
        _ballq_body,
        grid=(B, S // SBLK),
        in_specs=[
            pl.BlockSpec((1, 3, N), lambda b, sb: (b, 0, 0)),
            pl.BlockSpec((1, SBLK, 3), lambda b, sb: (b, sb, 0)),
        ],
        out_specs=pl.BlockSpec((1, SBLK, K), lambda b, sb: (b, sb, 0)),
        out_shape=jax.ShapeDtypeStruct((B, S, K), jnp.int32),
        compiler_params=pltpu.CompilerParams(
            dimension_semantics=("parallel", "parallel")),
    )(point_xyz, nxt)


CHP = 4096


def _pts_body(t_ref, w_ref, o_ref):
    o_ref[...] = jnp.dot(t_ref[...], w_ref[...], precision=HIGH,
                         preferred_element_type=jnp.float32)


def _pts_call(table6, w1):
    return pl.pallas_call(
        _pts_body,
        grid=(B * N // CHP,),
        in_specs=[
            pl.BlockSpec((CHP, 6), lambda i: (i, 0)),
            pl.BlockSpec((6, 32), lambda i: (0, 0)),
        ],
        out_specs=pl.BlockSpec((CHP, 32), lambda i: (i, 0)),
        out_shape=jax.ShapeDtypeStruct((B * N, 32), jnp.float32),
        compiler_params=pltpu.CompilerParams(
            dimension_semantics=("parallel",)),
    )(table6, w1)


def _cen_body(nx_ref, w_ref, b_ref, o_ref):
    wx = w_ref[0:3, :]
    q = jax.lax.dot_general(wx, nx_ref[...], (((0,), (0,)), ((), ())),
                            precision=HIGH,
                            preferred_element_type=jnp.float32)
    o_ref[...] = b_ref[...] - q


def _cen_call(nx3m, w1, b1col):
    return pl.pallas_call(
        _cen_body,
        in_specs=[
            pl.BlockSpec((3, M), lambda: (0, 0)),
            pl.BlockSpec((6, 32), lambda: (0, 0)),
            pl.BlockSpec((32, 1), lambda: (0, 0)),
        ],
        grid=(),
        out_specs=pl.BlockSpec((32, M), lambda: (0, 0)),
        out_shape=jax.ShapeDtypeStruct((32, M), jnp.float32),
    )(nx3m, w1, b1col)


NW = 32
ROWS_W = NTOT // NW
GCH = 1024


def _sc_gather(table, idx_flat):
    mesh = plsc.VectorSubcoreMesh(core_axis_name="c", subcore_axis_name="s")

    @functools.partial(
        pl.kernel,
        out_type=jax.ShapeDtypeStruct((NTOT, 32), jnp.float32),
        mesh=mesh,
        scratch_types=[
            pltpu.VMEM((GCH,), jnp.int32),
            pltpu.VMEM((GCH, 32), jnp.float32),
            pltpu.SemaphoreType.DMA,
        ],
        compiler_params=pltpu.CompilerParams(use_tc_tiling_on_sc=False),
    )
    def k(table_hbm, idx_hbm, out_hbm, idx_v, rows_v, sem):
        wid = jax.lax.axis_index("s") * 2 + jax.lax.axis_index("c")

        @pl.loop(0, ROWS_W // GCH)
        def _(ci):
            base = wid * ROWS_W + ci * GCH
            pltpu.sync_copy(idx_hbm.at[pl.ds(base, GCH)], idx_v)
            pltpu.async_copy(table_hbm.at[idx_v], rows_v, sem).wait()
            pltpu.sync_copy(rows_v, out_hbm.at[pl.ds(base, GCH)])

    return k(table, idx_flat)


def _bn_cols(st_ref, g_ref, be_ref):
    mean = st_ref[:, 0:1] * np.float32(1.0 / NTOT)
    var = st_ref[:, 1:2] * np.float32(1.0 / NTOT) - mean * mean
    scale = g_ref[...] * jax.lax.rsqrt(var + EPS)
    shift = be_ref[...] - mean * scale
    return scale, shift


def _b1_body(g_ref, q_ref, y_ref, st_ref, acc):
    pid = pl.program_id(0)

    @pl.when(pid == 0)
    def _():
        acc[...] = jnp.zeros_like(acc)

    eye = (jax.lax.broadcasted_iota(jnp.int32, (32, 32), 0)
           == jax.lax.broadcasted_iota(jnp.int32, (32, 32), 1)
           ).astype(jnp.float32)
    gt = jax.lax.dot_general(eye, g_ref[0], (((1,), (1,)), ((), ())),
                             precision=HIGH,
                             preferred_element_type=jnp.float32)
    y = gt + q_ref[...]
    acc[:, 0:1] = acc[:, 0:1] + jnp.sum(y, axis=1, keepdims=True)
    acc[:, 1:2] = acc[:, 1:2] + jnp.sum(y * y, axis=1, keepdims=True)
    y_ref[0] = y

    @pl.when(pid == pl.num_programs(0) - 1)
    def _():
        st_ref[...] = acc[...]


def _b1_call(g3d, q1t):
    return pl.pallas_call(
        _b1_body,
        grid=(K,),
        in_specs=[
            pl.BlockSpec((1, M, 32), lambda k: (k, 0, 0)),
            pl.BlockSpec((32, M), lambda k: (0, 0)),
        ],
        out_specs=[
            pl.BlockSpec((1, 32, M), lambda k: (k, 0, 0)),
            pl.BlockSpec((32, 2), lambda k: (0, 0)),
        ],
        out_shape=[
            jax.ShapeDtypeStruct((K, 32, M), jnp.float32),
            jax.ShapeDtypeStruct((32, 2), jnp.float32),
        ],
        scratch_shapes=[pltpu.VMEM((32, 2), jnp.float32)],
    )(g3d, q1t)


def _mid_body(cin, cout, y_ref, st_ref, g_ref, be_ref, w_ref, b_ref,
              o_ref, so_ref, acc):
    pid = pl.program_id(0)

    @pl.when(pid == 0)
    def _():
        acc[...] = jnp.zeros_like(acc)

    scale, shift = _bn_cols(st_ref, g_ref, be_ref)
    h = jnp.maximum(y_ref[0] * scale + shift, 0.0)
    y2 = jax.lax.dot_general(w_ref[...], h, (((0,), (0,)), ((), ())),
                             precision=HIGH,
                             preferred_element_type=jnp.float32) + b_ref[...]
    acc[:, 0:1] = acc[:, 0:1] + jnp.sum(y2, axis=1, keepdims=True)
    acc[:, 1:2] = acc[:, 1:2] + jnp.sum(y2 * y2, axis=1, keepdims=True)
    o_ref[0] = y2

    @pl.when(pid == pl.num_programs(0) - 1)
    def _():
        so_ref[...] = acc[...]


def _mid_call(y, st, g, be, w, b, cin, cout):
    return pl.pallas_call(
        functools.partial(_mid_body, cin, cout),
        grid=(K,),
        in_specs=[
            pl.BlockSpec((1, cin, M), lambda k: (k, 0, 0)),
            pl.BlockSpec((cin, 2), lambda k: (0, 0)),
            pl.BlockSpec((cin, 1), lambda k: (0, 0)),
            pl.BlockSpec((cin, 1), lambda k: (0, 0)),
            pl.BlockSpec((cin, cout), lambda k: (0, 0)),
            pl.BlockSpec((cout, 1), lambda k: (0, 0)),
        ],
        out_specs=[
            pl.BlockSpec((1, cout, M), lambda k: (k, 0, 0)),
            pl.BlockSpec((cout, 2), lambda k: (0, 0)),
        ],
        out_shape=[
            jax.ShapeDtypeStruct((K, cout, M), jnp.float32),
            jax.ShapeDtypeStruct((cout, 2), jnp.float32),
        ],
        scratch_shapes=[pltpu.VMEM((cout, 2), jnp.float32)],
    )(y, st, g, be, w, b)


CHL = 1024


def _fin_body(y_ref, st_ref, g_ref, be_ref, o_ref):
    scale, shift = _bn_cols(st_ref, g_ref, be_ref)
    h = jnp.maximum(y_ref[...] * scale[None, :, :] + shift[None, :, :], 0.0)
    o_ref[...] = jnp.max(h, axis=0)


def _fin_call(y, st, g, be):
    return pl.pallas_call(
        _fin_body,
        grid=(M // CHL,),
        in_specs=[
            pl.BlockSpec((K, 64, CHL), lambda i: (0, 0, i)),
            pl.BlockSpec((64, 2), lambda i: (0, 0)),
            pl.BlockSpec((64, 1), lambda i: (0, 0)),
            pl.BlockSpec((64, 1), lambda i: (0, 0)),
        ],
        out_specs=pl.BlockSpec((64, CHL), lambda i: (0, i)),
        out_shape=jax.ShapeDtypeStruct((64, M), jnp.float32),
        compiler_params=pltpu.CompilerParams(
            dimension_semantics=("parallel",)),
    )(y, st, g, be)


def kernel(point_xyz, point_features, W1, b1, g1, be1,
           W2, b2, g2, be2, W3, b3, g3, be3):
    new_xyz = _fps_call(point_xyz)
    nxt = new_xyz.transpose(0, 2, 1)
    idx = _ballq_call(point_xyz, nxt)
    idx_flat = idx.transpose(2, 0, 1).reshape(NTOT)

    xyz_rows = point_xyz.transpose(0, 2, 1).reshape(B * N, 3)
    feat_rows = point_features.transpose(0, 2, 1).reshape(B * N, 3)
    table6 = jnp.concatenate([xyz_rows, feat_rows], axis=1)

    p1 = _pts_call(table6, W1)
    nx3m = new_xyz.transpose(1, 0, 2).reshape(3, M)
    q1t = _cen_call(nx3m, W1, b1.reshape(32, 1))

    g = _sc_gather(p1, idx_flat).reshape(K, M, 32)

    y1, st1 = _b1_call(g, q1t)
    y2, st2 = _mid_call(y1, st1, g1.reshape(32, 1), be1.reshape(32, 1),
                        W2, b2.reshape(32, 1), 32, 32)
    y3, st3 = _mid_call(y2, st2, g2.reshape(32, 1), be2.reshape(32, 1),
                        W3, b3.reshape(64, 1), 32, 64)
    hm = _fin_call(y3, st3, g3.reshape(64, 1), be3.reshape(64, 1))

    feat_out = hm.reshape(64, B, S).transpose(1, 0, 2)
    return new_xyz, feat_out


# speedup vs baseline: 1.2301x; 1.2301x over previous
import functools
import numpy as np
import jax
import jax.numpy as jnp
from jax.experimental import pallas as pl
from jax.experimental.pallas import tpu as pltpu
from jax.experimental.pallas import tpu_sc as plsc

B, N, S, K = 8, 4096, 1024, 32
R2 = np.float32(np.float64(0.2) ** 2)
M = B * S
NTOT = K * M
EPS = np.float32(1e-5)
SBLK = 256
HIGH = jax.lax.Precision.HIGHEST


def _fps_body(xyz_ref, out_ref):
    x = xyz_ref[:, 0, :]
    y = xyz_ref[:, 1, :]
    z = xyz_ref[:, 2, :]
    lane_n = jax.lax.broadcasted_iota(jnp.int32, (B, N), 1)
    lane_s = jax.lax.broadcasted_iota(jnp.int32, (B, S), 1)

    def step(t, carry):
        dist, far, ax, ay, az = carry
        oh = (lane_n == far).astype(jnp.float32)
        cx = jnp.sum(x * oh, axis=1, keepdims=True)
        cy = jnp.sum(y * oh, axis=1, keepdims=True)
        cz = jnp.sum(z * oh, axis=1, keepdims=True)
        ohs = (lane_s == t).astype(jnp.float32)
        ax = ax + cx * ohs
        ay = ay + cy * ohs
        az = az + cz * ohs
        dx = x - cx
        dy = y - cy
        dz = z - cz
        d = dx * dx + dy * dy + dz * dz
        dist = jnp.minimum(dist, d)
        m = jnp.max(dist, axis=1, keepdims=True)
        far = jnp.min(jnp.where(dist == m, lane_n, N), axis=1, keepdims=True)
        return dist, far, ax, ay, az

    dist0 = jnp.full((B, N), 1e10, jnp.float32)
    far0 = jnp.zeros((B, 1), jnp.int32)
    zS = jnp.zeros((B, S), jnp.float32)
    _, _, ax, ay, az = jax.lax.fori_loop(0, S, step, (dist0, far0, zS, zS, zS))
    out_ref[:, 0, :] = ax
    out_ref[:, 1, :] = ay
    out_ref[:, 2, :] = az


def _fps_call(point_xyz):
    return pl.pallas_call(
        _fps_body,
        out_shape=jax.ShapeDtypeStruct((B, 3, S), jnp.float32),
    )(point_xyz)


def _ballq_body(xyz_ref, nxt_ref, out_ref):
    b = pl.program_id(0)
    sb = pl.program_id(1)
    px = xyz_ref[0]
    cn = nxt_ref[0]
    cc = jnp.sum(cn * cn, axis=1, keepdims=True)
    pp = jnp.sum(px * px, axis=0, keepdims=True)
    mm = jnp.dot(cn, px, preferred_element_type=jnp.float32)
    d2 = (cc + pp) - 2.0 * mm
    mask = (d2 <= R2).astype(jnp.bfloat16)
    nch = N // 128
    li = jax.lax.broadcasted_iota(jnp.int32, (128, 128), 0)
    lj = jax.lax.broadcasted_iota(jnp.int32, (128, 128), 1)
    ut = (li <= lj).astype(jnp.bfloat16)
    intras = [jnp.dot(mask[:, c * 128:(c + 1) * 128], ut,
                      preferred_element_type=jnp.float32)
              for c in range(nch)]
    cs = jnp.concatenate([ic[:, 127:128] for ic in intras], axis=1)
    ci = jax.lax.broadcasted_iota(jnp.int32, (nch, nch), 0)
    cj = jax.lax.broadcasted_iota(jnp.int32, (nch, nch), 1)
    uts = (ci < cj).astype(jnp.float32)
    csx = jnp.dot(cs, uts, precision=HIGH,
                  preferred_element_type=jnp.float32)
    rank = jnp.concatenate(
        [intras[c] + csx[:, c:c + 1] for c in range(nch)], axis=1)
    count = rank[:, N - 1:N]
    raws = [jnp.sum((rank <= j).astype(jnp.float32), axis=1, keepdims=True)
            for j in range(K)]
    raw = jnp.concatenate(raws, axis=1).astype(jnp.int32)
    count = count.astype(jnp.int32)
    jv = jax.lax.broadcasted_iota(jnp.int32, (SBLK, K), 1)
    idx = jnp.where(jv < count, raw, raw[:, 0:1]) + b * N
    ki = jax.lax.broadcasted_iota(jnp.int32, (K, K), 0)
    kj = jax.lax.broadcasted_iota(jnp.int32, (K, K), 1)
    eyek = (ki == kj).astype(jnp.float32)
    idx_t = jax.lax.dot_general(eyek, idx.astype(jnp.float32),
                                (((1,), (1,)), ((), ())), precision=HIGH,
                                preferred_element_type=jnp.float32)
    out_ref[:, pl.ds(b, 1), pl.ds(sb * SBLK, SBLK)] = (
        idx_t.astype(jnp.int32)[:, None, :])


def _ballq_call(point_xyz, nxt):
    return pl.pallas_call(
        _ballq_body,
        grid=(B, S // SBLK),
        in_specs=[
            pl.BlockSpec((1, 3, N), lambda b, sb: (b, 0, 0)),
            pl.BlockSpec((1, SBLK, 3), lambda b, sb: (b, sb, 0)),
        ],
        out_specs=pl.BlockSpec((K, B, S), lambda b, sb: (0, 0, 0)),
        out_shape=jax.ShapeDtypeStruct((K, B, S), jnp.int32),
    )(point_xyz, nxt)


def _pts_body(xyz_ref, ft_ref, w_ref, o_ref):
    px = xyz_ref[0]
    pf = ft_ref[0]
    dn = (((0,), (0,)), ((), ()))
    p = (jax.lax.dot_general(px, w_ref[0:3, :], dn, precision=HIGH,
                             preferred_element_type=jnp.float32)
         + jax.lax.dot_general(pf, w_ref[3:6, :], dn, precision=HIGH,
                               preferred_element_type=jnp.float32))
    o_ref[...] = p


def _pts_call(point_xyz, point_features, w1):
    return pl.pallas_call(
        _pts_body,
        grid=(B,),
        in_specs=[
            pl.BlockSpec((1, 3, N), lambda b: (b, 0, 0)),
            pl.BlockSpec((1, 3, N), lambda b: (b, 0, 0)),
            pl.BlockSpec((6, 32), lambda b: (0, 0)),
        ],
        out_specs=pl.BlockSpec((N, 32), lambda b: (b, 0)),
        out_shape=jax.ShapeDtypeStruct((B * N, 32), jnp.float32),
    )(point_xyz, point_features, w1)


def _cen_body(nx_ref, w_ref, b_ref, o_ref):
    wx = w_ref[0:3, :]
    q = jax.lax.dot_general(wx, nx_ref[...], (((0,), (0,)), ((), ())),
                            precision=HIGH,
                            preferred_element_type=jnp.float32)
    o_ref[...] = b_ref[...] - q


def _cen_call(nx3m, w1, b1col):
    return pl.pallas_call(
        _cen_body,
        in_specs=[
            pl.BlockSpec((3, M), lambda: (0, 0)),
            pl.BlockSpec((6, 32), lambda: (0, 0)),
            pl.BlockSpec((32, 1), lambda: (0, 0)),
        ],
        grid=(),
        out_specs=pl.BlockSpec((32, M), lambda: (0, 0)),
        out_shape=jax.ShapeDtypeStruct((32, M), jnp.float32),
    )(nx3m, w1, b1col)


NW = 32
ROWS_W = NTOT // NW
GCH = 1024


def _sc_gather(table, idx_flat):
    mesh = plsc.VectorSubcoreMesh(core_axis_name="c", subcore_axis_name="s")

    @functools.partial(
        pl.kernel,
        out_type=jax.ShapeDtypeStruct((NTOT, 32), jnp.float32),
        mesh=mesh,
        scratch_types=[
            pltpu.VMEM((GCH,), jnp.int32),
            pltpu.VMEM((GCH, 32), jnp.float32),
            pltpu.SemaphoreType.DMA,
        ],
        compiler_params=pltpu.CompilerParams(use_tc_tiling_on_sc=False),
    )
    def k(table_hbm, idx_hbm, out_hbm, idx_v, rows_v, sem):
        wid = jax.lax.axis_index("s") * 2 + jax.lax.axis_index("c")

        @pl.loop(0, ROWS_W // GCH)
        def _(ci):
            base = wid * ROWS_W + ci * GCH
            pltpu.sync_copy(idx_hbm.at[pl.ds(base, GCH)], idx_v)
            pltpu.async_copy(table_hbm.at[idx_v], rows_v, sem).wait()
            pltpu.sync_copy(rows_v, out_hbm.at[pl.ds(base, GCH)])

    return k(table, idx_flat)


def _bn_cols(st_ref, g_ref, be_ref):
    mean = st_ref[:, 0:1] * np.float32(1.0 / NTOT)
    var = st_ref[:, 1:2] * np.float32(1.0 / NTOT) - mean * mean
    scale = g_ref[...] * jax.lax.rsqrt(var + EPS)
    shift = be_ref[...] - mean * scale
    return scale, shift


def _b1_body(g_ref, q_ref, y_ref, st_ref, acc):
    pid = pl.program_id(0)

    @pl.when(pid == 0)
    def _():
        acc[...] = jnp.zeros_like(acc)

    eye = (jax.lax.broadcasted_iota(jnp.int32, (32, 32), 0)
           == jax.lax.broadcasted_iota(jnp.int32, (32, 32), 1)
           ).astype(jnp.float32)
    gt = jax.lax.dot_general(eye, g_ref[0], (((1,), (1,)), ((), ())),
                             precision=HIGH,
                             preferred_element_type=jnp.float32)
    y = gt + q_ref[...]
    acc[:, 0:1] = acc[:, 0:1] + jnp.sum(y, axis=1, keepdims=True)
    acc[:, 1:2] = acc[:, 1:2] + jnp.sum(y * y, axis=1, keepdims=True)
    y_ref[0] = y

    @pl.when(pid == pl.num_programs(0) - 1)
    def _():
        st_ref[...] = acc[...]


def _b1_call(g3d, q1t):
    return pl.pallas_call(
        _b1_body,
        grid=(K,),
        in_specs=[
            pl.BlockSpec((1, M, 32), lambda k: (k, 0, 0)),
            pl.BlockSpec((32, M), lambda k: (0, 0)),
        ],
        out_specs=[
            pl.BlockSpec((1, 32, M), lambda k: (k, 0, 0)),
            pl.BlockSpec((32, 2), lambda k: (0, 0)),
        ],
        out_shape=[
            jax.ShapeDtypeStruct((K, 32, M), jnp.float32),
            jax.ShapeDtypeStruct((32, 2), jnp.float32),
        ],
        scratch_shapes=[pltpu.VMEM((32, 2), jnp.float32)],
    )(g3d, q1t)


def _mid_body(cin, cout, y_ref, st_ref, g_ref, be_ref, w_ref, b_ref,
              o_ref, so_ref, acc):
    pid = pl.program_id(0)

    @pl.when(pid == 0)
    def _():
        acc[...] = jnp.zeros_like(acc)

    scale, shift = _bn_cols(st_ref, g_ref, be_ref)
    h = jnp.maximum(y_ref[0] * scale + shift, 0.0)
    y2 = jax.lax.dot_general(w_ref[...], h, (((0,), (0,)), ((), ())),
                             precision=HIGH,
                             preferred_element_type=jnp.float32) + b_ref[...]
    acc[:, 0:1] = acc[:, 0:1] + jnp.sum(y2, axis=1, keepdims=True)
    acc[:, 1:2] = acc[:, 1:2] + jnp.sum(y2 * y2, axis=1, keepdims=True)
    o_ref[0] = y2

    @pl.when(pid == pl.num_programs(0) - 1)
    def _():
        so_ref[...] = acc[...]


def _mid_call(y, st, g, be, w, b, cin, cout):
    return pl.pallas_call(
        functools.partial(_mid_body, cin, cout),
        grid=(K,),
        in_specs=[
            pl.BlockSpec((1, cin, M), lambda k: (k, 0, 0)),
            pl.BlockSpec((cin, 2), lambda k: (0, 0)),
            pl.BlockSpec((cin, 1), lambda k: (0, 0)),
            pl.BlockSpec((cin, 1), lambda k: (0, 0)),
            pl.BlockSpec((cin, cout), lambda k: (0, 0)),
            pl.BlockSpec((cout, 1), lambda k: (0, 0)),
        ],
        out_specs=[
            pl.BlockSpec((1, cout, M), lambda k: (k, 0, 0)),
            pl.BlockSpec((cout, 2), lambda k: (0, 0)),
        ],
        out_shape=[
            jax.ShapeDtypeStruct((K, cout, M), jnp.float32),
            jax.ShapeDtypeStruct((cout, 2), jnp.float32),
        ],
        scratch_shapes=[pltpu.VMEM((cout, 2), jnp.float32)],
    )(y, st, g, be, w, b)


CHL = 1024


def _fin_body(y_ref, st_ref, g_ref, be_ref, o_ref):
    scale, shift = _bn_cols(st_ref, g_ref, be_ref)
    h = jnp.maximum(y_ref[...] * scale[None, :, :] + shift[None, :, :], 0.0)
    o_ref[0] = jnp.max(h, axis=0)


def _fin_call(y, st, g, be):
    return pl.pallas_call(
        _fin_body,
        grid=(M // CHL,),
        in_specs=[
            pl.BlockSpec((K, 64, CHL), lambda i: (0, 0, i)),
            pl.BlockSpec((64, 2), lambda i: (0, 0)),
            pl.BlockSpec((64, 1), lambda i: (0, 0)),
            pl.BlockSpec((64, 1), lambda i: (0, 0)),
        ],
        out_specs=pl.BlockSpec((1, 64, CHL), lambda i: (i, 0, 0)),
        out_shape=jax.ShapeDtypeStruct((B, 64, S), jnp.float32),
    )(y, st, g, be)


def kernel(point_xyz, point_features, W1, b1, g1, be1,
           W2, b2, g2, be2, W3, b3, g3, be3):
    new_xyz = _fps_call(point_xyz)
    nxt = new_xyz.transpose(0, 2, 1)
    idx_t = _ballq_call(point_xyz, nxt)
    idx_flat = idx_t.reshape(NTOT)

    p1 = _pts_call(point_xyz, point_features, W1)
    nx3m = new_xyz.transpose(1, 0, 2).reshape(3, M)
    q1t = _cen_call(nx3m, W1, b1.reshape(32, 1))

    g = _sc_gather(p1, idx_flat).reshape(K, M, 32)

    y1, st1 = _b1_call(g, q1t)
    y2, st2 = _mid_call(y1, st1, g1.reshape(32, 1), be1.reshape(32, 1),
                        W2, b2.reshape(32, 1), 32, 32)
    y3, st3 = _mid_call(y2, st2, g2.reshape(32, 1), be2.reshape(32, 1),
                        W3, b3.reshape(64, 1), 32, 64)
    feat_out = _fin_call(y3, st3, g3.reshape(64, 1), be3.reshape(64, 1))
    return new_xyz, feat_out
